# baseline (speedup 1.0000x reference)
import functools

import jax
import jax.numpy as jnp
from jax import lax
from jax.experimental import pallas as pl
from jax.experimental.pallas import tpu as pltpu
from jax.experimental.pallas import tpu_sc as plsc

_BATCH_NUM = 256
_BATCH_SZ = 4096
_NODE = 64

_NC = 2
_NS = 16
_NW = _NC * _NS
_SUBROW = 512
_SUBROWS_PER_BATCH = _BATCH_SZ * _NODE // _SUBROW
_ROWS_PER_W = _SUBROWS_PER_BATCH // _NW

_mesh = plsc.VectorSubcoreMesh(core_axis_name="c", subcore_axis_name="s")


@functools.partial(
    pl.kernel,
    out_type=jax.ShapeDtypeStruct((_SUBROWS_PER_BATCH, _SUBROW), jnp.float32),
    mesh=_mesh,
    scratch_types=[
        pltpu.VMEM((_ROWS_PER_W,), jnp.int32),
        pltpu.VMEM((_ROWS_PER_W, _SUBROW), jnp.float32),
        pltpu.SemaphoreType.DMA,
    ],
)
def _gather_row(table_hbm, idx_hbm, out_hbm, idx_v, buf, sem):
    wid = lax.axis_index("s") * _NC + lax.axis_index("c")
    pltpu.sync_copy(idx_hbm.at[wid], idx_v)
    pltpu.async_copy(table_hbm.at[idx_v], buf, sem).wait()
    pltpu.sync_copy(buf, out_hbm.at[pl.ds(wid * _ROWS_PER_W, _ROWS_PER_W), :])


def kernel(para, batch_idx):
    table = para.reshape(_BATCH_NUM * _SUBROWS_PER_BATCH, _SUBROW)
    b = jnp.asarray(batch_idx, jnp.int32)
    idx = (b * _SUBROWS_PER_BATCH
           + jnp.arange(_SUBROWS_PER_BATCH, dtype=jnp.int32)).reshape(_NW, _ROWS_PER_W)
    out = _gather_row(table, idx)
    return out.reshape(_BATCH_SZ, _NODE, 1)


# speedup vs baseline: 1.6658x; 1.6658x over previous
import functools

import jax
import jax.numpy as jnp
from jax import lax
from jax.experimental import pallas as pl
from jax.experimental.pallas import tpu as pltpu
from jax.experimental.pallas import tpu_sc as plsc

_BATCH_NUM = 256
_BATCH_SZ = 4096
_NODE = 64

_NC = 2
_NS = 16
_NW = _NC * _NS
_ROWS_PER_W = _BATCH_SZ // _NW

_mesh = plsc.VectorSubcoreMesh(core_axis_name="c", subcore_axis_name="s")


@functools.partial(
    pl.kernel,
    out_type=jax.ShapeDtypeStruct((_BATCH_SZ, _NODE), jnp.float32),
    mesh=_mesh,
    scratch_types=[
        pltpu.VMEM((16,), jnp.int32),
        pltpu.VMEM((_ROWS_PER_W, _NODE), jnp.float32),
    ],
)
def _gather_row(para_hbm, idx_hbm, out_hbm, idx_v, buf):
    wid = lax.axis_index("s") * _NC + lax.axis_index("c")
    pltpu.sync_copy(idx_hbm, idx_v)
    b = idx_v[...][0]
    base = wid * _ROWS_PER_W
    pltpu.sync_copy(para_hbm.at[b, pl.ds(base, _ROWS_PER_W), :], buf)
    pltpu.sync_copy(buf, out_hbm.at[pl.ds(base, _ROWS_PER_W), :])


def kernel(para, batch_idx):
    idx = jnp.full((16,), batch_idx, dtype=jnp.int32)
    out = _gather_row(para, idx)
    return out[:, :, None]
